# Initial kernel scaffold; baseline (speedup 1.0000x reference)
import jax
import jax.numpy as jnp
from jax.experimental import pallas as pl


def kernel(pro1_x, pro1_edge_index, pro1_batch, pro2_x, pro2_edge_index, pro2_batch, W1, att_src1, att_dst1, b1, fc1_W, fc1_b, W2, att_src2, att_dst2, b2, fc2_W, fc2_b, final_W, final_b):
    raise NotImplementedError("write your pallas kernel here")


# speedup vs baseline: 13.4684x; 13.4684x over previous
import functools

import jax
import jax.numpy as jnp
from jax import lax
from jax.experimental import pallas as pl
from jax.experimental.pallas import tpu as pltpu
from jax.experimental.pallas import tpu_sc as plsc

N_NODES = 10000
N_EDGES = 160000
D_FEAT = 256
HIDDEN = 128
OUT_DIM = 128
NUM_GRAPHS = 32

NPAD = 10240
ROW_TILE = 512
N_ROW_TILES = NPAD // ROW_TILE
EPT = N_EDGES // 16
EPT_PAD = 10240
N_CHUNKS = EPT_PAD // 128
SLAB = NPAD // 16


def _proj_body(x_ref, w_ref, att_ref, h_ref, a_ref):
    xb = x_ref[0]
    wb = w_ref[0]
    hb = jnp.dot(xb, wb, preferred_element_type=jnp.float32)
    h_ref[0] = hb
    a_ref[0] = jnp.dot(hb, att_ref[0], preferred_element_type=jnp.float32)


def _project(xs, Ws, atts):
    return pl.pallas_call(
        _proj_body,
        grid=(2, N_ROW_TILES),
        in_specs=[
            pl.BlockSpec((1, ROW_TILE, D_FEAT), lambda p, j: (p, j, 0)),
            pl.BlockSpec((1, D_FEAT, HIDDEN), lambda p, j: (p, 0, 0)),
            pl.BlockSpec((1, HIDDEN, 2), lambda p, j: (p, 0, 0)),
        ],
        out_specs=[
            pl.BlockSpec((1, ROW_TILE, HIDDEN), lambda p, j: (p, j, 0)),
            pl.BlockSpec((1, ROW_TILE, 2), lambda p, j: (p, j, 0)),
        ],
        out_shape=[
            jax.ShapeDtypeStruct((2, NPAD, HIDDEN), jnp.float32),
            jax.ShapeDtypeStruct((2, NPAD, 2), jnp.float32),
        ],
    )(xs, Ws, atts)


def _edge_body(h_hbm, asrc_hbm, adst_hbm, srcp_hbm, dstp_hbm,
               accum_out, denom_out,
               src_v, dst_v, w_v, idx_v, abufA, abufB, rowbuf,
               accum_sh, denom_sh, gsem):
    c = lax.axis_index("c")
    s = lax.axis_index("s")
    cN = c * NPAD

    def _zrow(r, _):
        for g in range(HIDDEN // 16):
            rowbuf[r, pl.ds(g * 16, 16)] = jnp.zeros((16,), jnp.float32)
        return _
    lax.fori_loop(0, 128, _zrow, None)
    for k in range(SLAB // 128):
        base = s * SLAB + k * 128
        pltpu.sync_copy(rowbuf, accum_sh.at[pl.ds(base, 128)])
        pltpu.sync_copy(rowbuf.at[0], denom_sh.at[pl.ds(base, 128)])
    plsc.subcore_barrier()

    pltpu.sync_copy(srcp_hbm.at[c, s], src_v)
    pltpu.sync_copy(dstp_hbm.at[c, s], dst_v)

    def _scal(r, _):
        for g in range(8):
            idx_v[pl.ds(g * 16, 16)] = src_v[r, pl.ds(g * 16, 16)] + cN
        pltpu.async_copy(asrc_hbm.at[idx_v], abufA, gsem).wait()
        for g in range(8):
            idx_v[pl.ds(g * 16, 16)] = dst_v[r, pl.ds(g * 16, 16)] + cN
        pltpu.async_copy(adst_hbm.at[idx_v], abufB, gsem).wait()
        for g in range(8):
            e = abufA[pl.ds(g * 16, 16)] + abufB[pl.ds(g * 16, 16)]
            e = jnp.maximum(e, 0.2 * e)
            w_v[r, pl.ds(g * 16, 16)] = jnp.exp(e)
        return _
    lax.fori_loop(0, N_CHUNKS, _scal, None)

    def _dns(r, _):
        pltpu.sync_copy(w_v.at[r], denom_sh.at[dst_v.at[r]], add=True)
        return _
    lax.fori_loop(0, N_CHUNKS, _dns, None)

    def _rows(r, _):
        for g in range(8):
            idx_v[pl.ds(g * 16, 16)] = src_v[r, pl.ds(g * 16, 16)] + cN
        pltpu.async_copy(h_hbm.at[idx_v], rowbuf, gsem).wait()

        def _scale(k, __):
            ir = jnp.full((16,), r, jnp.int32)
            ik = jnp.full((16,), k, jnp.int32)
            wsp = plsc.load_gather(w_v, [ir, ik])
            for g in range(HIDDEN // 16):
                rowbuf[k, pl.ds(g * 16, 16)] = rowbuf[k, pl.ds(g * 16, 16)] * wsp
            return __
        lax.fori_loop(0, 128, _scale, None)
        pltpu.sync_copy(rowbuf, accum_sh.at[dst_v.at[r]], add=True)
        return _
    lax.fori_loop(0, N_CHUNKS, _rows, None)

    plsc.subcore_barrier()

    for k in range(SLAB // 128):
        base = s * SLAB + k * 128
        pltpu.sync_copy(accum_sh.at[pl.ds(base, 128)], rowbuf)
        pltpu.sync_copy(rowbuf, accum_out.at[pl.ds(cN + base, 128)])
        pltpu.sync_copy(denom_sh.at[pl.ds(base, 128)], abufA)
        pltpu.sync_copy(abufA, denom_out.at[pl.ds(cN + base, 128)])


def _edge_phase(h_flat, asrc_flat, adst_flat, srcp, dstp):
    mesh = plsc.VectorSubcoreMesh(core_axis_name="c", subcore_axis_name="s")
    fn = functools.partial(
        pl.kernel,
        out_type=[
            jax.ShapeDtypeStruct((2 * NPAD, HIDDEN), jnp.float32),
            jax.ShapeDtypeStruct((2 * NPAD,), jnp.float32),
        ],
        mesh=mesh,
        compiler_params=pltpu.CompilerParams(needs_layout_passes=False),
        scratch_types=[
            pltpu.VMEM((N_CHUNKS, 128), jnp.int32),
            pltpu.VMEM((N_CHUNKS, 128), jnp.int32),
            pltpu.VMEM((N_CHUNKS, 128), jnp.float32),
            pltpu.VMEM((128,), jnp.int32),
            pltpu.VMEM((128,), jnp.float32),
            pltpu.VMEM((128,), jnp.float32),
            pltpu.VMEM((128, HIDDEN), jnp.float32),
            pltpu.VMEM_SHARED((NPAD, HIDDEN), jnp.float32),
            pltpu.VMEM_SHARED((NPAD,), jnp.float32),
            pltpu.SemaphoreType.DMA,
        ],
    )(_edge_body)
    return fn(h_flat, asrc_flat, adst_flat, srcp, dstp)


def _head_body(acc_ref, den_ref, bat_ref, bias_ref, fcw_ref, fcb_ref,
               finw_ref, finb_ref, out_ref, pooled, cnt, ys):
    p = pl.program_id(0)
    j = pl.program_id(1)

    @pl.when(j == 0)
    def _():
        pooled[...] = jnp.zeros_like(pooled)
        cnt[...] = jnp.zeros_like(cnt)

    acc = acc_ref[0]
    den = den_ref[0]
    xb = acc / jnp.maximum(den, 1e-16) + bias_ref[0]
    xb = jnp.maximum(xb, 0.01 * xb)
    bt = bat_ref[0]
    gid = lax.broadcasted_iota(jnp.int32, (ROW_TILE, NUM_GRAPHS), 1
                               ).astype(jnp.float32)
    onehot = (bt == gid).astype(jnp.float32)
    pooled[...] += lax.dot_general(onehot, xb, (((0,), (0,)), ((), ())),
                                   preferred_element_type=jnp.float32)
    cnt[...] += jnp.sum(onehot, axis=0)[:, None]

    @pl.when(j == N_ROW_TILES - 1)
    def _():
        mean = pooled[...] / jnp.maximum(cnt[...], 1.0)
        y = jnp.dot(mean, fcw_ref[0], preferred_element_type=jnp.float32)
        y = y + fcb_ref[0]
        y = jnp.maximum(y, 0.01 * y)
        ys[:, pl.ds(p * OUT_DIM, OUT_DIM)] = y

    @pl.when((p == 1) & (j == N_ROW_TILES - 1))
    def _():
        out_ref[...] = (jnp.dot(ys[...], finw_ref[...],
                                preferred_element_type=jnp.float32)
                        + finb_ref[...])


def _heads(accum, denom, batch_f, biases, fcWs, fcbs, finW, finb):
    return pl.pallas_call(
        _head_body,
        grid=(2, N_ROW_TILES),
        in_specs=[
            pl.BlockSpec((1, ROW_TILE, HIDDEN), lambda p, j: (p, j, 0)),
            pl.BlockSpec((1, ROW_TILE, 1), lambda p, j: (p, j, 0)),
            pl.BlockSpec((1, ROW_TILE, 1), lambda p, j: (p, j, 0)),
            pl.BlockSpec((1, 1, HIDDEN), lambda p, j: (p, 0, 0)),
            pl.BlockSpec((1, HIDDEN, OUT_DIM), lambda p, j: (p, 0, 0)),
            pl.BlockSpec((1, 1, OUT_DIM), lambda p, j: (p, 0, 0)),
            pl.BlockSpec((2 * OUT_DIM, 1), lambda p, j: (0, 0)),
            pl.BlockSpec((1, 1), lambda p, j: (0, 0)),
        ],
        out_specs=pl.BlockSpec((NUM_GRAPHS, 1), lambda p, j: (0, 0)),
        out_shape=jax.ShapeDtypeStruct((NUM_GRAPHS, 1), jnp.float32),
        scratch_shapes=[
            pltpu.VMEM((NUM_GRAPHS, OUT_DIM), jnp.float32),
            pltpu.VMEM((NUM_GRAPHS, OUT_DIM), jnp.float32),
            pltpu.VMEM((NUM_GRAPHS, 2 * OUT_DIM), jnp.float32),
        ],
    )(accum, denom, batch_f, biases, fcWs, fcbs, finW, finb)


def _pad_edges(ei):
    r = ei.astype(jnp.int32).reshape(16, EPT)
    r = jnp.pad(r, ((0, 0), (0, EPT_PAD - EPT)), constant_values=NPAD - 1)
    return r.reshape(16, N_CHUNKS, 128)


def kernel(pro1_x, pro1_edge_index, pro1_batch, pro2_x, pro2_edge_index,
           pro2_batch, W1, att_src1, att_dst1, b1, fc1_W, fc1_b, W2,
           att_src2, att_dst2, b2, fc2_W, fc2_b, final_W, final_b):
    pad_n = NPAD - N_NODES
    xs = jnp.stack([
        jnp.pad(pro1_x, ((0, pad_n), (0, 0))),
        jnp.pad(pro2_x, ((0, pad_n), (0, 0))),
    ])
    Ws = jnp.stack([W1, W2])
    atts = jnp.stack([
        jnp.stack([att_src1, att_dst1], axis=1),
        jnp.stack([att_src2, att_dst2], axis=1),
    ])
    h, a = _project(xs, Ws, atts)
    h_flat = h.reshape(2 * NPAD, HIDDEN)
    asrc_flat = a[:, :, 0].reshape(2 * NPAD)
    adst_flat = a[:, :, 1].reshape(2 * NPAD)

    srcp = jnp.stack([_pad_edges(pro1_edge_index[0]),
                      _pad_edges(pro2_edge_index[0])])
    dstp = jnp.stack([_pad_edges(pro1_edge_index[1]),
                      _pad_edges(pro2_edge_index[1])])

    accum_flat, denom_flat = _edge_phase(h_flat, asrc_flat, adst_flat,
                                         srcp, dstp)
    accum = accum_flat.reshape(2, NPAD, HIDDEN)
    denom = denom_flat.reshape(2, NPAD, 1)

    batch_f = jnp.stack([
        jnp.pad(pro1_batch.astype(jnp.int32), (0, pad_n),
                constant_values=NUM_GRAPHS),
        jnp.pad(pro2_batch.astype(jnp.int32), (0, pad_n),
                constant_values=NUM_GRAPHS),
    ]).astype(jnp.float32).reshape(2, NPAD, 1)

    biases = jnp.stack([b1, b2]).reshape(2, 1, HIDDEN)
    fcWs = jnp.stack([fc1_W, fc2_W])
    fcbs = jnp.stack([fc1_b, fc2_b]).reshape(2, 1, OUT_DIM)

    return _heads(accum, denom, batch_f, biases, fcWs, fcbs,
                  final_W, final_b.reshape(1, 1))


# speedup vs baseline: 15.5458x; 1.1542x over previous
import functools

import jax
import jax.numpy as jnp
from jax import lax
from jax.experimental import pallas as pl
from jax.experimental.pallas import tpu as pltpu
from jax.experimental.pallas import tpu_sc as plsc

N_NODES = 10000
N_EDGES = 160000
D_FEAT = 256
HIDDEN = 128
OUT_DIM = 128
NUM_GRAPHS = 32

NPAD = 10240
ROW_TILE = 512
N_ROW_TILES = NPAD // ROW_TILE
EPT = N_EDGES // 16
EPT_PAD = 10240
N_CHUNKS = EPT_PAD // 128
SLAB = NPAD // 16


def _proj_body(x_ref, w_ref, att_ref, h_ref, a_ref):
    xb = x_ref[0]
    wb = w_ref[0]
    hb = jnp.dot(xb, wb, preferred_element_type=jnp.float32)
    h_ref[0] = hb
    a_ref[0] = jnp.dot(hb, att_ref[0], preferred_element_type=jnp.float32)


def _project(xs, Ws, atts):
    return pl.pallas_call(
        _proj_body,
        grid=(2, N_ROW_TILES),
        in_specs=[
            pl.BlockSpec((1, ROW_TILE, D_FEAT), lambda p, j: (p, j, 0)),
            pl.BlockSpec((1, D_FEAT, HIDDEN), lambda p, j: (p, 0, 0)),
            pl.BlockSpec((1, HIDDEN, 2), lambda p, j: (p, 0, 0)),
        ],
        out_specs=[
            pl.BlockSpec((1, ROW_TILE, HIDDEN), lambda p, j: (p, j, 0)),
            pl.BlockSpec((1, ROW_TILE, 2), lambda p, j: (p, j, 0)),
        ],
        out_shape=[
            jax.ShapeDtypeStruct((2, NPAD, HIDDEN), jnp.float32),
            jax.ShapeDtypeStruct((2, NPAD, 2), jnp.float32),
        ],
    )(xs, Ws, atts)


def _edge_body(h_hbm, asrc_hbm, adst_hbm, srcp_hbm, dstp_hbm,
               accum_out, denom_out,
               srccb0, srccb1, dstcb0, dstcb1, idxh0, idxh1, idxd,
               abufA, abufB, wbuf0, wbuf1, rowbuf0, rowbuf1,
               accum_sh, denom_sh, hsem0, hsem1, asemA, asemB):
    c = lax.axis_index("c")
    s = lax.axis_index("s")
    cN = c * NPAD
    srccb = (srccb0, srccb1)
    dstcb = (dstcb0, dstcb1)
    idxh = (idxh0, idxh1)
    wbuf = (wbuf0, wbuf1)
    rowbuf = (rowbuf0, rowbuf1)
    hsem = (hsem0, hsem1)

    def _zrow(r, _):
        for g in range(HIDDEN // 16):
            rowbuf0[r, pl.ds(g * 16, 16)] = jnp.zeros((16,), jnp.float32)
        return _
    lax.fori_loop(0, 128, _zrow, None)
    for k in range(SLAB // 128):
        base = s * SLAB + k * 128
        pltpu.sync_copy(rowbuf0, accum_sh.at[pl.ds(base, 128)])
        pltpu.sync_copy(rowbuf0.at[0], denom_sh.at[pl.ds(base, 128)])
    plsc.subcore_barrier()

    def _load_chunk(ch, b):
        pltpu.sync_copy(srcp_hbm.at[c, s, ch], srccb[b])
        pltpu.sync_copy(dstp_hbm.at[c, s, ch], dstcb[b])
        for g in range(8):
            idxh[b][pl.ds(g * 16, 16)] = srccb[b][pl.ds(g * 16, 16)] + cN
        pltpu.async_copy(h_hbm.at[idxh[b]], rowbuf[b], hsem[b])

    _load_chunk(0, 0)

    def _pair(i, _):
        for b in range(2):
            ch = i * 2 + b
            b2 = 1 - b

            @pl.when(ch + 1 < N_CHUNKS)
            def _():
                _load_chunk(ch + 1, b2)

            cpA = pltpu.async_copy(asrc_hbm.at[idxh[b]], abufA, asemA)
            for g in range(8):
                idxd[pl.ds(g * 16, 16)] = dstcb[b][pl.ds(g * 16, 16)] + cN
            cpB = pltpu.async_copy(adst_hbm.at[idxd], abufB, asemB)
            cpA.wait()
            cpB.wait()
            for g in range(8):
                e = abufA[pl.ds(g * 16, 16)] + abufB[pl.ds(g * 16, 16)]
                e = jnp.maximum(e, 0.2 * e)
                wbuf[b][pl.ds(g * 16, 16)] = jnp.exp(e)
            pltpu.sync_copy(wbuf[b], denom_sh.at[dstcb[b]], add=True)

            pltpu.make_async_copy(h_hbm.at[idxh[b]], rowbuf[b], hsem[b]).wait()

            def _scale(k, __):
                ik = jnp.full((16,), k, jnp.int32)
                wsp = plsc.load_gather(wbuf[b], [ik])
                for g in range(HIDDEN // 16):
                    rowbuf[b][k, pl.ds(g * 16, 16)] = (
                        rowbuf[b][k, pl.ds(g * 16, 16)] * wsp)
                return __
            lax.fori_loop(0, 128, _scale, None)
            pltpu.sync_copy(rowbuf[b], accum_sh.at[dstcb[b]], add=True)
        return _
    lax.fori_loop(0, N_CHUNKS // 2, _pair, None)

    plsc.subcore_barrier()

    for k in range(SLAB // 128):
        base = s * SLAB + k * 128
        pltpu.sync_copy(accum_sh.at[pl.ds(base, 128)], rowbuf0)
        pltpu.sync_copy(rowbuf0, accum_out.at[pl.ds(cN + base, 128)])
        pltpu.sync_copy(denom_sh.at[pl.ds(base, 128)], abufA)
        pltpu.sync_copy(abufA, denom_out.at[pl.ds(cN + base, 128)])


def _edge_phase(h_flat, asrc_flat, adst_flat, srcp, dstp):
    mesh = plsc.VectorSubcoreMesh(core_axis_name="c", subcore_axis_name="s")
    fn = functools.partial(
        pl.kernel,
        out_type=[
            jax.ShapeDtypeStruct((2 * NPAD, HIDDEN), jnp.float32),
            jax.ShapeDtypeStruct((2 * NPAD,), jnp.float32),
        ],
        mesh=mesh,
        compiler_params=pltpu.CompilerParams(needs_layout_passes=False),
        scratch_types=[
            pltpu.VMEM((128,), jnp.int32),
            pltpu.VMEM((128,), jnp.int32),
            pltpu.VMEM((128,), jnp.int32),
            pltpu.VMEM((128,), jnp.int32),
            pltpu.VMEM((128,), jnp.int32),
            pltpu.VMEM((128,), jnp.int32),
            pltpu.VMEM((128,), jnp.int32),
            pltpu.VMEM((128,), jnp.float32),
            pltpu.VMEM((128,), jnp.float32),
            pltpu.VMEM((128,), jnp.float32),
            pltpu.VMEM((128,), jnp.float32),
            pltpu.VMEM((128, HIDDEN), jnp.float32),
            pltpu.VMEM((128, HIDDEN), jnp.float32),
            pltpu.VMEM_SHARED((NPAD, HIDDEN), jnp.float32),
            pltpu.VMEM_SHARED((NPAD,), jnp.float32),
            pltpu.SemaphoreType.DMA,
            pltpu.SemaphoreType.DMA,
            pltpu.SemaphoreType.DMA,
            pltpu.SemaphoreType.DMA,
        ],
    )(_edge_body)
    return fn(h_flat, asrc_flat, adst_flat, srcp, dstp)


def _head_body(acc_ref, den_ref, bat_ref, bias_ref, fcw_ref, fcb_ref,
               finw_ref, finb_ref, out_ref, pooled, cnt, ys):
    p = pl.program_id(0)
    j = pl.program_id(1)

    @pl.when(j == 0)
    def _():
        pooled[...] = jnp.zeros_like(pooled)
        cnt[...] = jnp.zeros_like(cnt)

    acc = acc_ref[0]
    den = den_ref[0]
    xb = acc / jnp.maximum(den, 1e-16) + bias_ref[0]
    xb = jnp.maximum(xb, 0.01 * xb)
    bt = bat_ref[0]
    gid = lax.broadcasted_iota(jnp.int32, (ROW_TILE, NUM_GRAPHS), 1
                               ).astype(jnp.float32)
    onehot = (bt == gid).astype(jnp.float32)
    pooled[...] += lax.dot_general(onehot, xb, (((0,), (0,)), ((), ())),
                                   preferred_element_type=jnp.float32,
                                   precision=lax.Precision.HIGHEST)
    cnt[...] += jnp.sum(onehot, axis=0)[:, None]

    @pl.when(j == N_ROW_TILES - 1)
    def _():
        mean = pooled[...] / jnp.maximum(cnt[...], 1.0)
        y = jnp.dot(mean, fcw_ref[0], preferred_element_type=jnp.float32,
                    precision=lax.Precision.HIGHEST)
        y = y + fcb_ref[0]
        y = jnp.maximum(y, 0.01 * y)
        ys[:, pl.ds(p * OUT_DIM, OUT_DIM)] = y

    @pl.when((p == 1) & (j == N_ROW_TILES - 1))
    def _():
        out_ref[...] = (jnp.dot(ys[...], finw_ref[...],
                                preferred_element_type=jnp.float32,
                                precision=lax.Precision.HIGHEST)
                        + finb_ref[...])


def _heads(accum, denom, batch_f, biases, fcWs, fcbs, finW, finb):
    return pl.pallas_call(
        _head_body,
        grid=(2, N_ROW_TILES),
        in_specs=[
            pl.BlockSpec((1, ROW_TILE, HIDDEN), lambda p, j: (p, j, 0)),
            pl.BlockSpec((1, ROW_TILE, 1), lambda p, j: (p, j, 0)),
            pl.BlockSpec((1, ROW_TILE, 1), lambda p, j: (p, j, 0)),
            pl.BlockSpec((1, 1, HIDDEN), lambda p, j: (p, 0, 0)),
            pl.BlockSpec((1, HIDDEN, OUT_DIM), lambda p, j: (p, 0, 0)),
            pl.BlockSpec((1, 1, OUT_DIM), lambda p, j: (p, 0, 0)),
            pl.BlockSpec((2 * OUT_DIM, 1), lambda p, j: (0, 0)),
            pl.BlockSpec((1, 1), lambda p, j: (0, 0)),
        ],
        out_specs=pl.BlockSpec((NUM_GRAPHS, 1), lambda p, j: (0, 0)),
        out_shape=jax.ShapeDtypeStruct((NUM_GRAPHS, 1), jnp.float32),
        scratch_shapes=[
            pltpu.VMEM((NUM_GRAPHS, OUT_DIM), jnp.float32),
            pltpu.VMEM((NUM_GRAPHS, OUT_DIM), jnp.float32),
            pltpu.VMEM((NUM_GRAPHS, 2 * OUT_DIM), jnp.float32),
        ],
    )(accum, denom, batch_f, biases, fcWs, fcbs, finW, finb)


def _pad_edges(ei):
    r = ei.astype(jnp.int32).reshape(16, EPT)
    r = jnp.pad(r, ((0, 0), (0, EPT_PAD - EPT)), constant_values=NPAD - 1)
    return r.reshape(16, N_CHUNKS, 128)


def kernel(pro1_x, pro1_edge_index, pro1_batch, pro2_x, pro2_edge_index,
           pro2_batch, W1, att_src1, att_dst1, b1, fc1_W, fc1_b, W2,
           att_src2, att_dst2, b2, fc2_W, fc2_b, final_W, final_b):
    pad_n = NPAD - N_NODES
    xs = jnp.stack([
        jnp.pad(pro1_x, ((0, pad_n), (0, 0))),
        jnp.pad(pro2_x, ((0, pad_n), (0, 0))),
    ])
    Ws = jnp.stack([W1, W2])
    atts = jnp.stack([
        jnp.stack([att_src1, att_dst1], axis=1),
        jnp.stack([att_src2, att_dst2], axis=1),
    ])
    h, a = _project(xs, Ws, atts)
    h_flat = h.reshape(2 * NPAD, HIDDEN)
    asrc_flat = a[:, :, 0].reshape(2 * NPAD)
    adst_flat = a[:, :, 1].reshape(2 * NPAD)

    srcp = jnp.stack([_pad_edges(pro1_edge_index[0]),
                      _pad_edges(pro2_edge_index[0])])
    dstp = jnp.stack([_pad_edges(pro1_edge_index[1]),
                      _pad_edges(pro2_edge_index[1])])

    accum_flat, denom_flat = _edge_phase(h_flat, asrc_flat, adst_flat,
                                         srcp, dstp)
    accum = accum_flat.reshape(2, NPAD, HIDDEN)
    denom = denom_flat.reshape(2, NPAD, 1)

    batch_f = jnp.stack([
        jnp.pad(pro1_batch.astype(jnp.int32), (0, pad_n),
                constant_values=NUM_GRAPHS),
        jnp.pad(pro2_batch.astype(jnp.int32), (0, pad_n),
                constant_values=NUM_GRAPHS),
    ]).astype(jnp.float32).reshape(2, NPAD, 1)

    biases = jnp.stack([b1, b2]).reshape(2, 1, HIDDEN)
    fcWs = jnp.stack([fc1_W, fc2_W])
    fcbs = jnp.stack([fc1_b, fc2_b]).reshape(2, 1, OUT_DIM)

    return _heads(accum, denom, batch_f, biases, fcWs, fcbs,
                  final_W, final_b.reshape(1, 1))


# speedup vs baseline: 18.9155x; 1.2168x over previous
import functools

import jax
import jax.numpy as jnp
from jax import lax
from jax.experimental import pallas as pl
from jax.experimental.pallas import tpu as pltpu
from jax.experimental.pallas import tpu_sc as plsc

N_NODES = 10000
N_EDGES = 160000
D_FEAT = 256
HIDDEN = 128
OUT_DIM = 128
NUM_GRAPHS = 32

NPAD = 10240
ROW_TILE = 512
N_ROW_TILES = NPAD // ROW_TILE
EPT = N_EDGES // 16
EPT_PAD = 10240
N_CHUNKS = EPT_PAD // 128
SLAB = NPAD // 16


def _proj_body(x_ref, w_ref, att_ref, h_ref, a_ref):
    xb = x_ref[0]
    wb = w_ref[0]
    hb = jnp.dot(xb, wb, preferred_element_type=jnp.float32)
    h_ref[0] = hb
    a_ref[0] = jnp.dot(hb, att_ref[0], preferred_element_type=jnp.float32)


def _project(xs, Ws, atts):
    return pl.pallas_call(
        _proj_body,
        grid=(2, N_ROW_TILES),
        in_specs=[
            pl.BlockSpec((1, ROW_TILE, D_FEAT), lambda p, j: (p, j, 0)),
            pl.BlockSpec((1, D_FEAT, HIDDEN), lambda p, j: (p, 0, 0)),
            pl.BlockSpec((1, HIDDEN, 2), lambda p, j: (p, 0, 0)),
        ],
        out_specs=[
            pl.BlockSpec((1, ROW_TILE, HIDDEN), lambda p, j: (p, j, 0)),
            pl.BlockSpec((1, ROW_TILE, 2), lambda p, j: (p, j, 0)),
        ],
        out_shape=[
            jax.ShapeDtypeStruct((2, NPAD, HIDDEN), jnp.float32),
            jax.ShapeDtypeStruct((2, NPAD, 2), jnp.float32),
        ],
    )(xs, Ws, atts)


def _edge_body(h_hbm, asrc_hbm, adst_hbm, srcp_hbm, dstp_hbm,
               accum_out, denom_out,
               srccb0, srccb1, dstcb0, dstcb1, idxh0, idxh1, idxd0, idxd1,
               abufA0, abufA1, abufB0, abufB1, wbuf0, wbuf1,
               rowbuf0, rowbuf1, accum_sh, denom_sh,
               hsem0, hsem1, asemA0, asemA1, asemB0, asemB1, ssem0, ssem1):
    c = lax.axis_index("c")
    s = lax.axis_index("s")
    cN = c * NPAD
    srccb = (srccb0, srccb1)
    dstcb = (dstcb0, dstcb1)
    idxh = (idxh0, idxh1)
    idxd = (idxd0, idxd1)
    abufA = (abufA0, abufA1)
    abufB = (abufB0, abufB1)
    wbuf = (wbuf0, wbuf1)
    rowbuf = (rowbuf0, rowbuf1)
    hsem = (hsem0, hsem1)
    asemA = (asemA0, asemA1)
    asemB = (asemB0, asemB1)
    ssem = (ssem0, ssem1)

    def _zrow(r, _):
        for g in range(HIDDEN // 16):
            rowbuf0[r, pl.ds(g * 16, 16)] = jnp.zeros((16,), jnp.float32)
        return _
    lax.fori_loop(0, 128, _zrow, None)
    for k in range(SLAB // 128):
        base = s * SLAB + k * 128
        pltpu.sync_copy(rowbuf0, accum_sh.at[pl.ds(base, 128)])
        pltpu.sync_copy(rowbuf0.at[0], denom_sh.at[pl.ds(base, 128)])
    plsc.subcore_barrier()

    def _load_chunk(ch, b):
        pltpu.sync_copy(srcp_hbm.at[c, s, ch], srccb[b])
        pltpu.sync_copy(dstp_hbm.at[c, s, ch], dstcb[b])
        for g in range(8):
            idxh[b][pl.ds(g * 16, 16)] = srccb[b][pl.ds(g * 16, 16)] + cN
            idxd[b][pl.ds(g * 16, 16)] = dstcb[b][pl.ds(g * 16, 16)] + cN
        pltpu.async_copy(h_hbm.at[idxh[b]], rowbuf[b], hsem[b])
        pltpu.async_copy(asrc_hbm.at[idxh[b]], abufA[b], asemA[b])
        pltpu.async_copy(adst_hbm.at[idxd[b]], abufB[b], asemB[b])

    _load_chunk(0, 0)

    def _pair(i, _):
        for b in range(2):
            ch = i * 2 + b
            b2 = 1 - b

            @pl.when(ch + 1 < N_CHUNKS)
            def _():
                @pl.when(ch >= 1)
                def _():
                    pltpu.make_async_copy(
                        rowbuf[b2], accum_sh.at[dstcb[b2]], ssem[b2]).wait()
                _load_chunk(ch + 1, b2)

            pltpu.make_async_copy(
                asrc_hbm.at[idxh[b]], abufA[b], asemA[b]).wait()
            pltpu.make_async_copy(
                adst_hbm.at[idxd[b]], abufB[b], asemB[b]).wait()
            for g in range(8):
                e = abufA[b][pl.ds(g * 16, 16)] + abufB[b][pl.ds(g * 16, 16)]
                e = jnp.maximum(e, 0.2 * e)
                wbuf[b][pl.ds(g * 16, 16)] = jnp.exp(e)
            pltpu.sync_copy(wbuf[b], denom_sh.at[dstcb[b]], add=True)

            pltpu.make_async_copy(h_hbm.at[idxh[b]], rowbuf[b], hsem[b]).wait()

            @plsc.parallel_loop(0, 128, unroll=8)
            def _scale(k):
                ik = jnp.full((16,), k, jnp.int32)
                wsp = plsc.load_gather(wbuf[b], [ik])
                for g in range(HIDDEN // 16):
                    rowbuf[b][k, pl.ds(g * 16, 16)] = (
                        rowbuf[b][k, pl.ds(g * 16, 16)] * wsp)
            pltpu.async_copy(rowbuf[b], accum_sh.at[dstcb[b]], ssem[b],
                             add=True)
        return _
    lax.fori_loop(0, N_CHUNKS // 2, _pair, None)

    pltpu.make_async_copy(rowbuf[0], accum_sh.at[dstcb[0]], ssem[0]).wait()
    pltpu.make_async_copy(rowbuf[1], accum_sh.at[dstcb[1]], ssem[1]).wait()
    plsc.subcore_barrier()

    for k in range(SLAB // 128):
        base = s * SLAB + k * 128
        pltpu.sync_copy(accum_sh.at[pl.ds(base, 128)], rowbuf0)
        pltpu.sync_copy(rowbuf0, accum_out.at[pl.ds(cN + base, 128)])
        pltpu.sync_copy(denom_sh.at[pl.ds(base, 128)], abufA0)
        pltpu.sync_copy(abufA0, denom_out.at[pl.ds(cN + base, 128)])


def _edge_phase(h_flat, asrc_flat, adst_flat, srcp, dstp):
    mesh = plsc.VectorSubcoreMesh(core_axis_name="c", subcore_axis_name="s")
    fn = functools.partial(
        pl.kernel,
        out_type=[
            jax.ShapeDtypeStruct((2 * NPAD, HIDDEN), jnp.float32),
            jax.ShapeDtypeStruct((2 * NPAD,), jnp.float32),
        ],
        mesh=mesh,
        compiler_params=pltpu.CompilerParams(needs_layout_passes=False),
        scratch_types=[
            pltpu.VMEM((128,), jnp.int32),
            pltpu.VMEM((128,), jnp.int32),
            pltpu.VMEM((128,), jnp.int32),
            pltpu.VMEM((128,), jnp.int32),
            pltpu.VMEM((128,), jnp.int32),
            pltpu.VMEM((128,), jnp.int32),
            pltpu.VMEM((128,), jnp.int32),
            pltpu.VMEM((128,), jnp.int32),
            pltpu.VMEM((128,), jnp.float32),
            pltpu.VMEM((128,), jnp.float32),
            pltpu.VMEM((128,), jnp.float32),
            pltpu.VMEM((128,), jnp.float32),
            pltpu.VMEM((128,), jnp.float32),
            pltpu.VMEM((128,), jnp.float32),
            pltpu.VMEM((128, HIDDEN), jnp.float32),
            pltpu.VMEM((128, HIDDEN), jnp.float32),
            pltpu.VMEM_SHARED((NPAD, HIDDEN), jnp.float32),
            pltpu.VMEM_SHARED((NPAD,), jnp.float32),
            pltpu.SemaphoreType.DMA,
            pltpu.SemaphoreType.DMA,
            pltpu.SemaphoreType.DMA,
            pltpu.SemaphoreType.DMA,
            pltpu.SemaphoreType.DMA,
            pltpu.SemaphoreType.DMA,
            pltpu.SemaphoreType.DMA,
            pltpu.SemaphoreType.DMA,
        ],
    )(_edge_body)
    return fn(h_flat, asrc_flat, adst_flat, srcp, dstp)


def _head_body(acc_ref, den_ref, bat_ref, bias_ref, fcw_ref, fcb_ref,
               finw_ref, finb_ref, out_ref, pooled, cnt, ys):
    p = pl.program_id(0)
    j = pl.program_id(1)

    @pl.when(j == 0)
    def _():
        pooled[...] = jnp.zeros_like(pooled)
        cnt[...] = jnp.zeros_like(cnt)

    acc = acc_ref[0]
    den = den_ref[0]
    xb = acc / jnp.maximum(den, 1e-16) + bias_ref[0]
    xb = jnp.maximum(xb, 0.01 * xb)
    bt = bat_ref[0]
    gid = lax.broadcasted_iota(jnp.int32, (ROW_TILE, NUM_GRAPHS), 1
                               ).astype(jnp.float32)
    onehot = (bt == gid).astype(jnp.float32)
    pooled[...] += lax.dot_general(onehot, xb, (((0,), (0,)), ((), ())),
                                   preferred_element_type=jnp.float32,
                                   precision=lax.Precision.HIGHEST)
    cnt[...] += jnp.sum(onehot, axis=0)[:, None]

    @pl.when(j == N_ROW_TILES - 1)
    def _():
        mean = pooled[...] / jnp.maximum(cnt[...], 1.0)
        y = jnp.dot(mean, fcw_ref[0], preferred_element_type=jnp.float32,
                    precision=lax.Precision.HIGHEST)
        y = y + fcb_ref[0]
        y = jnp.maximum(y, 0.01 * y)
        ys[:, pl.ds(p * OUT_DIM, OUT_DIM)] = y

    @pl.when((p == 1) & (j == N_ROW_TILES - 1))
    def _():
        out_ref[...] = (jnp.dot(ys[...], finw_ref[...],
                                preferred_element_type=jnp.float32,
                                precision=lax.Precision.HIGHEST)
                        + finb_ref[...])


def _heads(accum, denom, batch_f, biases, fcWs, fcbs, finW, finb):
    return pl.pallas_call(
        _head_body,
        grid=(2, N_ROW_TILES),
        in_specs=[
            pl.BlockSpec((1, ROW_TILE, HIDDEN), lambda p, j: (p, j, 0)),
            pl.BlockSpec((1, ROW_TILE, 1), lambda p, j: (p, j, 0)),
            pl.BlockSpec((1, ROW_TILE, 1), lambda p, j: (p, j, 0)),
            pl.BlockSpec((1, 1, HIDDEN), lambda p, j: (p, 0, 0)),
            pl.BlockSpec((1, HIDDEN, OUT_DIM), lambda p, j: (p, 0, 0)),
            pl.BlockSpec((1, 1, OUT_DIM), lambda p, j: (p, 0, 0)),
            pl.BlockSpec((2 * OUT_DIM, 1), lambda p, j: (0, 0)),
            pl.BlockSpec((1, 1), lambda p, j: (0, 0)),
        ],
        out_specs=pl.BlockSpec((NUM_GRAPHS, 1), lambda p, j: (0, 0)),
        out_shape=jax.ShapeDtypeStruct((NUM_GRAPHS, 1), jnp.float32),
        scratch_shapes=[
            pltpu.VMEM((NUM_GRAPHS, OUT_DIM), jnp.float32),
            pltpu.VMEM((NUM_GRAPHS, OUT_DIM), jnp.float32),
            pltpu.VMEM((NUM_GRAPHS, 2 * OUT_DIM), jnp.float32),
        ],
    )(accum, denom, batch_f, biases, fcWs, fcbs, finW, finb)


def _pad_edges(ei):
    r = ei.astype(jnp.int32).reshape(16, EPT)
    r = jnp.pad(r, ((0, 0), (0, EPT_PAD - EPT)), constant_values=NPAD - 1)
    return r.reshape(16, N_CHUNKS, 128)


def kernel(pro1_x, pro1_edge_index, pro1_batch, pro2_x, pro2_edge_index,
           pro2_batch, W1, att_src1, att_dst1, b1, fc1_W, fc1_b, W2,
           att_src2, att_dst2, b2, fc2_W, fc2_b, final_W, final_b):
    pad_n = NPAD - N_NODES
    xs = jnp.stack([
        jnp.pad(pro1_x, ((0, pad_n), (0, 0))),
        jnp.pad(pro2_x, ((0, pad_n), (0, 0))),
    ])
    Ws = jnp.stack([W1, W2])
    atts = jnp.stack([
        jnp.stack([att_src1, att_dst1], axis=1),
        jnp.stack([att_src2, att_dst2], axis=1),
    ])
    h, a = _project(xs, Ws, atts)
    h_flat = h.reshape(2 * NPAD, HIDDEN)
    asrc_flat = a[:, :, 0].reshape(2 * NPAD)
    adst_flat = a[:, :, 1].reshape(2 * NPAD)

    srcp = jnp.stack([_pad_edges(pro1_edge_index[0]),
                      _pad_edges(pro2_edge_index[0])])
    dstp = jnp.stack([_pad_edges(pro1_edge_index[1]),
                      _pad_edges(pro2_edge_index[1])])

    accum_flat, denom_flat = _edge_phase(h_flat, asrc_flat, adst_flat,
                                         srcp, dstp)
    accum = accum_flat.reshape(2, NPAD, HIDDEN)
    denom = denom_flat.reshape(2, NPAD, 1)

    batch_f = jnp.stack([
        jnp.pad(pro1_batch.astype(jnp.int32), (0, pad_n),
                constant_values=NUM_GRAPHS),
        jnp.pad(pro2_batch.astype(jnp.int32), (0, pad_n),
                constant_values=NUM_GRAPHS),
    ]).astype(jnp.float32).reshape(2, NPAD, 1)

    biases = jnp.stack([b1, b2]).reshape(2, 1, HIDDEN)
    fcWs = jnp.stack([fc1_W, fc2_W])
    fcbs = jnp.stack([fc1_b, fc2_b]).reshape(2, 1, OUT_DIM)

    return _heads(accum, denom, batch_f, biases, fcWs, fcbs,
                  final_W, final_b.reshape(1, 1))


# speedup vs baseline: 21.5277x; 1.1381x over previous
import functools

import jax
import jax.numpy as jnp
from jax import lax
from jax.experimental import pallas as pl
from jax.experimental.pallas import tpu as pltpu
from jax.experimental.pallas import tpu_sc as plsc

N_NODES = 10000
N_EDGES = 160000
D_FEAT = 256
HIDDEN = 128
OUT_DIM = 128
NUM_GRAPHS = 32

NPAD = 10240
ROW_TILE = 512
N_ROW_TILES = NPAD // ROW_TILE
EPT = N_EDGES // 16
EPT_PAD = 10240
N_CHUNKS = EPT_PAD // 128
SLAB = NPAD // 16


def _proj_body(x_ref, w_ref, att_ref, h_ref, a_ref):
    xb = x_ref[0].astype(jnp.bfloat16)
    wb = w_ref[0].astype(jnp.bfloat16)
    hb = jnp.dot(xb, wb, preferred_element_type=jnp.float32)
    h_ref[0] = hb
    a_ref[0] = jnp.dot(hb.astype(jnp.bfloat16),
                       att_ref[0].astype(jnp.bfloat16),
                       preferred_element_type=jnp.float32)


def _project(xs, Ws, atts):
    return pl.pallas_call(
        _proj_body,
        grid=(2, N_ROW_TILES),
        in_specs=[
            pl.BlockSpec((1, ROW_TILE, D_FEAT), lambda p, j: (p, j, 0)),
            pl.BlockSpec((1, D_FEAT, HIDDEN), lambda p, j: (p, 0, 0)),
            pl.BlockSpec((1, HIDDEN, 2), lambda p, j: (p, 0, 0)),
        ],
        out_specs=[
            pl.BlockSpec((1, ROW_TILE, HIDDEN), lambda p, j: (p, j, 0)),
            pl.BlockSpec((1, ROW_TILE, 2), lambda p, j: (p, j, 0)),
        ],
        out_shape=[
            jax.ShapeDtypeStruct((2, NPAD, HIDDEN), jnp.float32),
            jax.ShapeDtypeStruct((2, NPAD, 2), jnp.float32),
        ],
    )(xs, Ws, atts)


BPC = 8
NBLK = N_CHUNKS // BPC


def _edge_body(h_hbm, asrc_hbm, adst_hbm, srcp_hbm, dstp_hbm,
               accum_out, denom_out,
               srcblk0, srcblk1, dstblk0, dstblk1, idxh0, idxh1, idxd0, idxd1,
               abufA0, abufA1, abufB0, abufB1, wbuf0, wbuf1,
               rowbuf0, rowbuf1, accum_sh, denom_sh,
               hsem0, hsem1, asemA0, asemA1, asemB0, asemB1,
               ssem0, ssem1, dsem0, dsem1, esemS0, esemS1, esemD0, esemD1):
    c = lax.axis_index("c")
    s = lax.axis_index("s")
    cN = c * NPAD
    srcblk = (srcblk0, srcblk1)
    dstblk = (dstblk0, dstblk1)
    idxh = (idxh0, idxh1)
    idxd = (idxd0, idxd1)
    abufA = (abufA0, abufA1)
    abufB = (abufB0, abufB1)
    wbuf = (wbuf0, wbuf1)
    rowbuf = (rowbuf0, rowbuf1)
    hsem = (hsem0, hsem1)
    asemA = (asemA0, asemA1)
    asemB = (asemB0, asemB1)
    ssem = (ssem0, ssem1)
    dsem = (dsem0, dsem1)
    esemS = (esemS0, esemS1)
    esemD = (esemD0, esemD1)

    def _zrow(r, _):
        for g in range(HIDDEN // 16):
            rowbuf0[r, pl.ds(g * 16, 16)] = jnp.zeros((16,), jnp.float32)
        return _
    lax.fori_loop(0, 128, _zrow, None)
    for k in range(SLAB // 128):
        base = s * SLAB + k * 128
        pltpu.sync_copy(rowbuf0, accum_sh.at[pl.ds(base, 128)])
        pltpu.sync_copy(rowbuf0.at[0], denom_sh.at[pl.ds(base, 128)])
    plsc.subcore_barrier()

    def _start_blk(j, bbuf):
        pltpu.async_copy(srcp_hbm.at[c, s, pl.ds(j * BPC, BPC)],
                         srcblk[bbuf], esemS[bbuf])
        pltpu.async_copy(dstp_hbm.at[c, s, pl.ds(j * BPC, BPC)],
                         dstblk[bbuf], esemD[bbuf])

    def _wait_blk(bbuf):
        pltpu.make_async_copy(srcp_hbm.at[c, s, pl.ds(0, BPC)],
                              srcblk[bbuf], esemS[bbuf]).wait()
        pltpu.make_async_copy(dstp_hbm.at[c, s, pl.ds(0, BPC)],
                              dstblk[bbuf], esemD[bbuf]).wait()

    def _drain_scatter(b):
        pltpu.make_async_copy(rowbuf[b], accum_sh.at[dstblk0.at[0]],
                              ssem[b]).wait()

    def _drain_denom(b):
        pltpu.make_async_copy(wbuf[b], denom_sh.at[dstblk0.at[0]],
                              dsem[b]).wait()

    def _prep(bbuf, k, b):
        for g in range(8):
            idxh[b][pl.ds(g * 16, 16)] = (
                srcblk[bbuf][k, pl.ds(g * 16, 16)] + cN)
            idxd[b][pl.ds(g * 16, 16)] = (
                dstblk[bbuf][k, pl.ds(g * 16, 16)] + cN)
        pltpu.async_copy(h_hbm.at[idxh[b]], rowbuf[b], hsem[b])
        pltpu.async_copy(asrc_hbm.at[idxh[b]], abufA[b], asemA[b])
        pltpu.async_copy(adst_hbm.at[idxd[b]], abufB[b], asemB[b])

    _start_blk(0, 0)
    _wait_blk(0)
    _prep(0, 0, 0)

    def _blockpair(i, _):
        for bb in range(2):
            jb = i * 2 + bb
            for k in range(BPC):
                b = k & 1
                b2 = 1 - b
                ch = jb * BPC + k

                if k < BPC - 1:
                    @pl.when(ch >= 1)
                    def _():
                        _drain_scatter(b2)
                    _prep(bb, k + 1, b2)
                else:
                    @pl.when(jb + 1 < NBLK)
                    def _():
                        _drain_scatter(b2)
                        _wait_blk(1 - bb)
                        _prep(1 - bb, 0, b2)

                if k == 2:
                    @pl.when(jb + 1 < NBLK)
                    def _():
                        _start_blk(jb + 1, 1 - bb)

                pltpu.make_async_copy(
                    asrc_hbm.at[idxh[b]], abufA[b], asemA[b]).wait()
                pltpu.make_async_copy(
                    adst_hbm.at[idxd[b]], abufB[b], asemB[b]).wait()

                @pl.when(ch >= 2)
                def _():
                    _drain_denom(b)
                for g in range(8):
                    e = (abufA[b][pl.ds(g * 16, 16)]
                         + abufB[b][pl.ds(g * 16, 16)])
                    e = jnp.maximum(e, 0.2 * e)
                    wbuf[b][pl.ds(g * 16, 16)] = jnp.exp(e)
                pltpu.async_copy(wbuf[b], denom_sh.at[dstblk[bb].at[k]],
                                 dsem[b], add=True)

                pltpu.make_async_copy(
                    h_hbm.at[idxh[b]], rowbuf[b], hsem[b]).wait()

                @plsc.parallel_loop(0, 128, unroll=8)
                def _scale(kk):
                    ik = jnp.full((16,), kk, jnp.int32)
                    wsp = plsc.load_gather(wbuf[b], [ik])
                    for g in range(HIDDEN // 16):
                        rowbuf[b][kk, pl.ds(g * 16, 16)] = (
                            rowbuf[b][kk, pl.ds(g * 16, 16)] * wsp)
                pltpu.async_copy(rowbuf[b], accum_sh.at[dstblk[bb].at[k]],
                                 ssem[b], add=True)
        return _
    lax.fori_loop(0, NBLK // 2, _blockpair, None)

    _drain_scatter(0)
    _drain_scatter(1)
    _drain_denom(0)
    _drain_denom(1)
    plsc.subcore_barrier()

    for k in range(SLAB // 128):
        base = s * SLAB + k * 128
        pltpu.sync_copy(accum_sh.at[pl.ds(base, 128)], rowbuf0)
        pltpu.sync_copy(rowbuf0, accum_out.at[pl.ds(cN + base, 128)])
        pltpu.sync_copy(denom_sh.at[pl.ds(base, 128)], abufA0)
        pltpu.sync_copy(abufA0, denom_out.at[pl.ds(cN + base, 128)])


def _edge_phase(h_flat, asrc_flat, adst_flat, srcp, dstp):
    mesh = plsc.VectorSubcoreMesh(core_axis_name="c", subcore_axis_name="s")
    fn = functools.partial(
        pl.kernel,
        out_type=[
            jax.ShapeDtypeStruct((2 * NPAD, HIDDEN), jnp.float32),
            jax.ShapeDtypeStruct((2 * NPAD,), jnp.float32),
        ],
        mesh=mesh,
        compiler_params=pltpu.CompilerParams(needs_layout_passes=False),
        scratch_types=[
            pltpu.VMEM((BPC, 128), jnp.int32),
            pltpu.VMEM((BPC, 128), jnp.int32),
            pltpu.VMEM((BPC, 128), jnp.int32),
            pltpu.VMEM((BPC, 128), jnp.int32),
            pltpu.VMEM((128,), jnp.int32),
            pltpu.VMEM((128,), jnp.int32),
            pltpu.VMEM((128,), jnp.int32),
            pltpu.VMEM((128,), jnp.int32),
            pltpu.VMEM((128,), jnp.float32),
            pltpu.VMEM((128,), jnp.float32),
            pltpu.VMEM((128,), jnp.float32),
            pltpu.VMEM((128,), jnp.float32),
            pltpu.VMEM((128,), jnp.float32),
            pltpu.VMEM((128,), jnp.float32),
            pltpu.VMEM((128, HIDDEN), jnp.float32),
            pltpu.VMEM((128, HIDDEN), jnp.float32),
            pltpu.VMEM_SHARED((NPAD, HIDDEN), jnp.float32),
            pltpu.VMEM_SHARED((NPAD,), jnp.float32),
            pltpu.SemaphoreType.DMA,
            pltpu.SemaphoreType.DMA,
            pltpu.SemaphoreType.DMA,
            pltpu.SemaphoreType.DMA,
            pltpu.SemaphoreType.DMA,
            pltpu.SemaphoreType.DMA,
            pltpu.SemaphoreType.DMA,
            pltpu.SemaphoreType.DMA,
            pltpu.SemaphoreType.DMA,
            pltpu.SemaphoreType.DMA,
            pltpu.SemaphoreType.DMA,
            pltpu.SemaphoreType.DMA,
            pltpu.SemaphoreType.DMA,
            pltpu.SemaphoreType.DMA,
        ],
    )(_edge_body)
    return fn(h_flat, asrc_flat, adst_flat, srcp, dstp)


def _head_body(acc_ref, den_ref, bat_ref, bias_ref, fcw_ref, fcb_ref,
               finw_ref, finb_ref, out_ref, pooled, cnt, ys):
    p = pl.program_id(0)
    j = pl.program_id(1)

    @pl.when(j == 0)
    def _():
        pooled[...] = jnp.zeros_like(pooled)
        cnt[...] = jnp.zeros_like(cnt)

    acc = acc_ref[0]
    den = den_ref[0]
    xb = acc / jnp.maximum(den, 1e-16) + bias_ref[0]
    xb = jnp.maximum(xb, 0.01 * xb)
    bt = bat_ref[0]
    gid = lax.broadcasted_iota(jnp.int32, (ROW_TILE, NUM_GRAPHS), 1
                               ).astype(jnp.float32)
    onehot = (bt == gid).astype(jnp.float32)
    pooled[...] += lax.dot_general(onehot, xb, (((0,), (0,)), ((), ())),
                                   preferred_element_type=jnp.float32,
                                   precision=lax.Precision.HIGHEST)
    cnt[...] += jnp.sum(onehot, axis=0)[:, None]

    @pl.when(j == N_ROW_TILES - 1)
    def _():
        mean = pooled[...] / jnp.maximum(cnt[...], 1.0)
        y = jnp.dot(mean.astype(jnp.bfloat16),
                    fcw_ref[0].astype(jnp.bfloat16),
                    preferred_element_type=jnp.float32)
        y = y + fcb_ref[0]
        y = jnp.maximum(y, 0.01 * y)
        ys[:, pl.ds(p * OUT_DIM, OUT_DIM)] = y

    @pl.when((p == 1) & (j == N_ROW_TILES - 1))
    def _():
        out_ref[...] = (jnp.dot(ys[...].astype(jnp.bfloat16),
                                finw_ref[...].astype(jnp.bfloat16),
                                preferred_element_type=jnp.float32)
                        + finb_ref[...])


def _heads(accum, denom, batch_f, biases, fcWs, fcbs, finW, finb):
    return pl.pallas_call(
        _head_body,
        grid=(2, N_ROW_TILES),
        in_specs=[
            pl.BlockSpec((1, ROW_TILE, HIDDEN), lambda p, j: (p, j, 0)),
            pl.BlockSpec((1, ROW_TILE, 1), lambda p, j: (p, j, 0)),
            pl.BlockSpec((1, ROW_TILE, 1), lambda p, j: (p, j, 0)),
            pl.BlockSpec((1, 1, HIDDEN), lambda p, j: (p, 0, 0)),
            pl.BlockSpec((1, HIDDEN, OUT_DIM), lambda p, j: (p, 0, 0)),
            pl.BlockSpec((1, 1, OUT_DIM), lambda p, j: (p, 0, 0)),
            pl.BlockSpec((2 * OUT_DIM, 1), lambda p, j: (0, 0)),
            pl.BlockSpec((1, 1), lambda p, j: (0, 0)),
        ],
        out_specs=pl.BlockSpec((NUM_GRAPHS, 1), lambda p, j: (0, 0)),
        out_shape=jax.ShapeDtypeStruct((NUM_GRAPHS, 1), jnp.float32),
        scratch_shapes=[
            pltpu.VMEM((NUM_GRAPHS, OUT_DIM), jnp.float32),
            pltpu.VMEM((NUM_GRAPHS, OUT_DIM), jnp.float32),
            pltpu.VMEM((NUM_GRAPHS, 2 * OUT_DIM), jnp.float32),
        ],
    )(accum, denom, batch_f, biases, fcWs, fcbs, finW, finb)


def _pad_edges(ei):
    r = ei.astype(jnp.int32).reshape(16, EPT)
    r = jnp.pad(r, ((0, 0), (0, EPT_PAD - EPT)), constant_values=NPAD - 1)
    return r.reshape(16, N_CHUNKS, 128)


def kernel(pro1_x, pro1_edge_index, pro1_batch, pro2_x, pro2_edge_index,
           pro2_batch, W1, att_src1, att_dst1, b1, fc1_W, fc1_b, W2,
           att_src2, att_dst2, b2, fc2_W, fc2_b, final_W, final_b):
    pad_n = NPAD - N_NODES
    xs = jnp.stack([
        jnp.pad(pro1_x, ((0, pad_n), (0, 0))),
        jnp.pad(pro2_x, ((0, pad_n), (0, 0))),
    ])
    Ws = jnp.stack([W1, W2])
    atts = jnp.stack([
        jnp.stack([att_src1, att_dst1], axis=1),
        jnp.stack([att_src2, att_dst2], axis=1),
    ])
    h, a = _project(xs, Ws, atts)
    h_flat = h.reshape(2 * NPAD, HIDDEN)
    asrc_flat = a[:, :, 0].reshape(2 * NPAD)
    adst_flat = a[:, :, 1].reshape(2 * NPAD)

    srcp = jnp.stack([_pad_edges(pro1_edge_index[0]),
                      _pad_edges(pro2_edge_index[0])])
    dstp = jnp.stack([_pad_edges(pro1_edge_index[1]),
                      _pad_edges(pro2_edge_index[1])])

    accum_flat, denom_flat = _edge_phase(h_flat, asrc_flat, adst_flat,
                                         srcp, dstp)
    accum = accum_flat.reshape(2, NPAD, HIDDEN)
    denom = denom_flat.reshape(2, NPAD, 1)

    batch_f = jnp.stack([
        jnp.pad(pro1_batch.astype(jnp.int32), (0, pad_n),
                constant_values=NUM_GRAPHS),
        jnp.pad(pro2_batch.astype(jnp.int32), (0, pad_n),
                constant_values=NUM_GRAPHS),
    ]).astype(jnp.float32).reshape(2, NPAD, 1)

    biases = jnp.stack([b1, b2]).reshape(2, 1, HIDDEN)
    fcWs = jnp.stack([fc1_W, fc2_W])
    fcbs = jnp.stack([fc1_b, fc2_b]).reshape(2, 1, OUT_DIM)

    return _heads(accum, denom, batch_f, biases, fcWs, fcbs,
                  final_W, final_b.reshape(1, 1))


# speedup vs baseline: 21.5805x; 1.0025x over previous
import functools

import jax
import jax.numpy as jnp
from jax import lax
from jax.experimental import pallas as pl
from jax.experimental.pallas import tpu as pltpu
from jax.experimental.pallas import tpu_sc as plsc

N_NODES = 10000
N_EDGES = 160000
D_FEAT = 256
HIDDEN = 128
OUT_DIM = 128
NUM_GRAPHS = 32

NPAD = 10240
ROW_TILE = 512
N_ROW_TILES = NPAD // ROW_TILE
EPT = N_EDGES // 16
EPT_PAD = 10240
N_CHUNKS = EPT_PAD // 128
SLAB = NPAD // 16


def _proj_body(x_ref, w_ref, att_ref, h_ref, a_ref):
    xb = x_ref[0].astype(jnp.bfloat16)
    wb = w_ref[0].astype(jnp.bfloat16)
    hb = jnp.dot(xb, wb, preferred_element_type=jnp.float32)
    h_ref[0] = hb
    a_ref[0] = jnp.dot(hb.astype(jnp.bfloat16),
                       att_ref[0].astype(jnp.bfloat16),
                       preferred_element_type=jnp.float32)


def _project(xs, Ws, atts):
    return pl.pallas_call(
        _proj_body,
        grid=(2, N_ROW_TILES),
        in_specs=[
            pl.BlockSpec((1, ROW_TILE, D_FEAT), lambda p, j: (p, j, 0)),
            pl.BlockSpec((1, D_FEAT, HIDDEN), lambda p, j: (p, 0, 0)),
            pl.BlockSpec((1, HIDDEN, 2), lambda p, j: (p, 0, 0)),
        ],
        out_specs=[
            pl.BlockSpec((1, ROW_TILE, HIDDEN), lambda p, j: (p, j, 0)),
            pl.BlockSpec((1, ROW_TILE, 2), lambda p, j: (p, j, 0)),
        ],
        out_shape=[
            jax.ShapeDtypeStruct((2, NPAD, HIDDEN), jnp.float32),
            jax.ShapeDtypeStruct((2, NPAD, 2), jnp.float32),
        ],
    )(xs, Ws, atts)


BPC = 8
NBLK = N_CHUNKS // BPC


def _edge_body(h_hbm, asrc_hbm, adst_hbm, srcp_hbm, dstp_hbm,
               accum_out, denom_out,
               srcblk0, srcblk1, dstblk0, dstblk1, idxh0, idxh1, idxd0, idxd1,
               abufA0, abufA1, abufB0, abufB1, wblk0, wblk1,
               rowbuf0, rowbuf1, accum_sh, denom_sh,
               hsem0, hsem1, asemA0, asemA1, asemB0, asemB1,
               ssem0, ssem1, dsem0, dsem1, esemS0, esemS1, esemD0, esemD1):
    c = lax.axis_index("c")
    s = lax.axis_index("s")
    cN = c * NPAD
    srcblk = (srcblk0, srcblk1)
    dstblk = (dstblk0, dstblk1)
    idxh = (idxh0, idxh1)
    idxd = (idxd0, idxd1)
    abufA = (abufA0, abufA1)
    abufB = (abufB0, abufB1)
    wblk = (wblk0, wblk1)
    rowbuf = (rowbuf0, rowbuf1)
    hsem = (hsem0, hsem1)
    asemA = (asemA0, asemA1)
    asemB = (asemB0, asemB1)
    ssem = (ssem0, ssem1)
    dsem = (dsem0, dsem1)
    esemS = (esemS0, esemS1)
    esemD = (esemD0, esemD1)

    def _zrow(r, _):
        for g in range(HIDDEN // 16):
            rowbuf0[r, pl.ds(g * 16, 16)] = jnp.zeros((16,), jnp.float32)
        return _
    lax.fori_loop(0, 128, _zrow, None)
    for k in range(SLAB // 128):
        base = s * SLAB + k * 128
        pltpu.sync_copy(rowbuf0, accum_sh.at[pl.ds(base, 128)])
        pltpu.sync_copy(rowbuf0.at[0], denom_sh.at[pl.ds(base, 128)])
    plsc.subcore_barrier()

    def _start_blk(j, bbuf):
        pltpu.async_copy(srcp_hbm.at[c, s, pl.ds(j * BPC, BPC)],
                         srcblk[bbuf], esemS[bbuf])
        pltpu.async_copy(dstp_hbm.at[c, s, pl.ds(j * BPC, BPC)],
                         dstblk[bbuf], esemD[bbuf])

    def _wait_blk(bbuf):
        pltpu.make_async_copy(srcp_hbm.at[c, s, pl.ds(0, BPC)],
                              srcblk[bbuf], esemS[bbuf]).wait()
        pltpu.make_async_copy(dstp_hbm.at[c, s, pl.ds(0, BPC)],
                              dstblk[bbuf], esemD[bbuf]).wait()

    def _drain_scatter(b):
        pltpu.make_async_copy(rowbuf[b], accum_sh.at[dstblk0.at[0]],
                              ssem[b]).wait()

    def _drain_denom(b):
        for k in range(BPC):
            pltpu.make_async_copy(wblk[b].at[k], denom_sh.at[dstblk0.at[0]],
                                  dsem[b]).wait()

    def _prep_a(bbuf, k, b):
        for g in range(8):
            idxh[b][pl.ds(g * 16, 16)] = (
                srcblk[bbuf][k, pl.ds(g * 16, 16)] + cN)
            idxd[b][pl.ds(g * 16, 16)] = (
                dstblk[bbuf][k, pl.ds(g * 16, 16)] + cN)
        pltpu.async_copy(asrc_hbm.at[idxh[b]], abufA[b], asemA[b])
        pltpu.async_copy(adst_hbm.at[idxd[b]], abufB[b], asemB[b])

    def _prep_h(b):
        pltpu.async_copy(h_hbm.at[idxh[b]], rowbuf[b], hsem[b])

    _start_blk(0, 0)
    _wait_blk(0)
    _prep_a(0, 0, 0)
    _prep_h(0)

    def _blockpair(i, _):
        for bb in range(2):
            jb = i * 2 + bb
            for k in range(BPC):
                b = k & 1
                b2 = 1 - b
                ch = jb * BPC + k

                if k < BPC - 1:
                    _prep_a(bb, k + 1, b2)

                    @pl.when(ch >= 1)
                    def _():
                        _drain_scatter(b2)
                    _prep_h(b2)
                else:
                    @pl.when(jb + 1 < NBLK)
                    def _():
                        _wait_blk(1 - bb)
                        _prep_a(1 - bb, 0, b2)
                        _drain_scatter(b2)
                        _prep_h(b2)

                if k == 2:
                    @pl.when(jb + 1 < NBLK)
                    def _():
                        @pl.when(jb >= 1)
                        def _():
                            _drain_denom(1 - bb)
                        _start_blk(jb + 1, 1 - bb)

                pltpu.make_async_copy(
                    asrc_hbm.at[idxh[b]], abufA[b], asemA[b]).wait()
                pltpu.make_async_copy(
                    adst_hbm.at[idxd[b]], abufB[b], asemB[b]).wait()
                for g in range(8):
                    e = (abufA[b][pl.ds(g * 16, 16)]
                         + abufB[b][pl.ds(g * 16, 16)])
                    e = jnp.maximum(e, 0.2 * e)
                    wblk[bb][k, pl.ds(g * 16, 16)] = jnp.exp(e)
                pltpu.async_copy(wblk[bb].at[k], denom_sh.at[dstblk[bb].at[k]],
                                 dsem[bb], add=True)

                pltpu.make_async_copy(
                    h_hbm.at[idxh[b]], rowbuf[b], hsem[b]).wait()

                ikc = jnp.full((16,), k, jnp.int32)

                @plsc.parallel_loop(0, 128, unroll=8)
                def _scale(kk):
                    ik = jnp.full((16,), kk, jnp.int32)
                    wsp = plsc.load_gather(wblk[bb], [ikc, ik])
                    for g in range(HIDDEN // 16):
                        rowbuf[b][kk, pl.ds(g * 16, 16)] = (
                            rowbuf[b][kk, pl.ds(g * 16, 16)] * wsp)
                pltpu.async_copy(rowbuf[b], accum_sh.at[dstblk[bb].at[k]],
                                 ssem[b], add=True)

        return _
    lax.fori_loop(0, NBLK // 2, _blockpair, None)

    _drain_scatter(0)
    _drain_scatter(1)
    _drain_denom(0)
    _drain_denom(1)
    plsc.subcore_barrier()

    for k in range(SLAB // 128):
        base = s * SLAB + k * 128
        pltpu.sync_copy(accum_sh.at[pl.ds(base, 128)], rowbuf0)
        pltpu.sync_copy(rowbuf0, accum_out.at[pl.ds(cN + base, 128)])
        pltpu.sync_copy(denom_sh.at[pl.ds(base, 128)], abufA0)
        pltpu.sync_copy(abufA0, denom_out.at[pl.ds(cN + base, 128)])


def _edge_phase(h_flat, asrc_flat, adst_flat, srcp, dstp):
    mesh = plsc.VectorSubcoreMesh(core_axis_name="c", subcore_axis_name="s")
    fn = functools.partial(
        pl.kernel,
        out_type=[
            jax.ShapeDtypeStruct((2 * NPAD, HIDDEN), jnp.float32),
            jax.ShapeDtypeStruct((2 * NPAD,), jnp.float32),
        ],
        mesh=mesh,
        compiler_params=pltpu.CompilerParams(needs_layout_passes=False),
        scratch_types=[
            pltpu.VMEM((BPC, 128), jnp.int32),
            pltpu.VMEM((BPC, 128), jnp.int32),
            pltpu.VMEM((BPC, 128), jnp.int32),
            pltpu.VMEM((BPC, 128), jnp.int32),
            pltpu.VMEM((128,), jnp.int32),
            pltpu.VMEM((128,), jnp.int32),
            pltpu.VMEM((128,), jnp.int32),
            pltpu.VMEM((128,), jnp.int32),
            pltpu.VMEM((128,), jnp.float32),
            pltpu.VMEM((128,), jnp.float32),
            pltpu.VMEM((128,), jnp.float32),
            pltpu.VMEM((128,), jnp.float32),
            pltpu.VMEM((BPC, 128), jnp.float32),
            pltpu.VMEM((BPC, 128), jnp.float32),
            pltpu.VMEM((128, HIDDEN), jnp.float32),
            pltpu.VMEM((128, HIDDEN), jnp.float32),
            pltpu.VMEM_SHARED((NPAD, HIDDEN), jnp.float32),
            pltpu.VMEM_SHARED((NPAD,), jnp.float32),
            pltpu.SemaphoreType.DMA,
            pltpu.SemaphoreType.DMA,
            pltpu.SemaphoreType.DMA,
            pltpu.SemaphoreType.DMA,
            pltpu.SemaphoreType.DMA,
            pltpu.SemaphoreType.DMA,
            pltpu.SemaphoreType.DMA,
            pltpu.SemaphoreType.DMA,
            pltpu.SemaphoreType.DMA,
            pltpu.SemaphoreType.DMA,
            pltpu.SemaphoreType.DMA,
            pltpu.SemaphoreType.DMA,
            pltpu.SemaphoreType.DMA,
            pltpu.SemaphoreType.DMA,
        ],
    )(_edge_body)
    return fn(h_flat, asrc_flat, adst_flat, srcp, dstp)


def _head_body(acc_ref, den_ref, bat_ref, bias_ref, fcw_ref, fcb_ref,
               finw_ref, finb_ref, out_ref, pooled, cnt, ys):
    p = pl.program_id(0)
    j = pl.program_id(1)

    @pl.when(j == 0)
    def _():
        pooled[...] = jnp.zeros_like(pooled)
        cnt[...] = jnp.zeros_like(cnt)

    acc = acc_ref[0]
    den = den_ref[0]
    xb = acc / jnp.maximum(den, 1e-16) + bias_ref[0]
    xb = jnp.maximum(xb, 0.01 * xb)
    bt = bat_ref[0]
    gid = lax.broadcasted_iota(jnp.int32, (ROW_TILE, NUM_GRAPHS), 1
                               ).astype(jnp.float32)
    onehot = (bt == gid).astype(jnp.float32)
    pooled[...] += lax.dot_general(onehot, xb, (((0,), (0,)), ((), ())),
                                   preferred_element_type=jnp.float32,
                                   precision=lax.Precision.HIGHEST)
    cnt[...] += jnp.sum(onehot, axis=0)[:, None]

    @pl.when(j == N_ROW_TILES - 1)
    def _():
        mean = pooled[...] / jnp.maximum(cnt[...], 1.0)
        y = jnp.dot(mean.astype(jnp.bfloat16),
                    fcw_ref[0].astype(jnp.bfloat16),
                    preferred_element_type=jnp.float32)
        y = y + fcb_ref[0]
        y = jnp.maximum(y, 0.01 * y)
        ys[:, pl.ds(p * OUT_DIM, OUT_DIM)] = y

    @pl.when((p == 1) & (j == N_ROW_TILES - 1))
    def _():
        out_ref[...] = (jnp.dot(ys[...].astype(jnp.bfloat16),
                                finw_ref[...].astype(jnp.bfloat16),
                                preferred_element_type=jnp.float32)
                        + finb_ref[...])


def _heads(accum, denom, batch_f, biases, fcWs, fcbs, finW, finb):
    return pl.pallas_call(
        _head_body,
        grid=(2, N_ROW_TILES),
        in_specs=[
            pl.BlockSpec((1, ROW_TILE, HIDDEN), lambda p, j: (p, j, 0)),
            pl.BlockSpec((1, ROW_TILE, 1), lambda p, j: (p, j, 0)),
            pl.BlockSpec((1, ROW_TILE, 1), lambda p, j: (p, j, 0)),
            pl.BlockSpec((1, 1, HIDDEN), lambda p, j: (p, 0, 0)),
            pl.BlockSpec((1, HIDDEN, OUT_DIM), lambda p, j: (p, 0, 0)),
            pl.BlockSpec((1, 1, OUT_DIM), lambda p, j: (p, 0, 0)),
            pl.BlockSpec((2 * OUT_DIM, 1), lambda p, j: (0, 0)),
            pl.BlockSpec((1, 1), lambda p, j: (0, 0)),
        ],
        out_specs=pl.BlockSpec((NUM_GRAPHS, 1), lambda p, j: (0, 0)),
        out_shape=jax.ShapeDtypeStruct((NUM_GRAPHS, 1), jnp.float32),
        scratch_shapes=[
            pltpu.VMEM((NUM_GRAPHS, OUT_DIM), jnp.float32),
            pltpu.VMEM((NUM_GRAPHS, OUT_DIM), jnp.float32),
            pltpu.VMEM((NUM_GRAPHS, 2 * OUT_DIM), jnp.float32),
        ],
    )(accum, denom, batch_f, biases, fcWs, fcbs, finW, finb)


def _pad_edges(ei):
    r = ei.astype(jnp.int32).reshape(16, EPT)
    r = jnp.pad(r, ((0, 0), (0, EPT_PAD - EPT)), constant_values=NPAD - 1)
    return r.reshape(16, N_CHUNKS, 128)


def kernel(pro1_x, pro1_edge_index, pro1_batch, pro2_x, pro2_edge_index,
           pro2_batch, W1, att_src1, att_dst1, b1, fc1_W, fc1_b, W2,
           att_src2, att_dst2, b2, fc2_W, fc2_b, final_W, final_b):
    pad_n = NPAD - N_NODES
    xs = jnp.stack([
        jnp.pad(pro1_x, ((0, pad_n), (0, 0))),
        jnp.pad(pro2_x, ((0, pad_n), (0, 0))),
    ])
    Ws = jnp.stack([W1, W2])
    atts = jnp.stack([
        jnp.stack([att_src1, att_dst1], axis=1),
        jnp.stack([att_src2, att_dst2], axis=1),
    ])
    h, a = _project(xs, Ws, atts)
    h_flat = h.reshape(2 * NPAD, HIDDEN)
    asrc_flat = a[:, :, 0].reshape(2 * NPAD)
    adst_flat = a[:, :, 1].reshape(2 * NPAD)

    srcp = jnp.stack([_pad_edges(pro1_edge_index[0]),
                      _pad_edges(pro2_edge_index[0])])
    dstp = jnp.stack([_pad_edges(pro1_edge_index[1]),
                      _pad_edges(pro2_edge_index[1])])

    accum_flat, denom_flat = _edge_phase(h_flat, asrc_flat, adst_flat,
                                         srcp, dstp)
    accum = accum_flat.reshape(2, NPAD, HIDDEN)
    denom = denom_flat.reshape(2, NPAD, 1)

    batch_f = jnp.stack([
        jnp.pad(pro1_batch.astype(jnp.int32), (0, pad_n),
                constant_values=NUM_GRAPHS),
        jnp.pad(pro2_batch.astype(jnp.int32), (0, pad_n),
                constant_values=NUM_GRAPHS),
    ]).astype(jnp.float32).reshape(2, NPAD, 1)

    biases = jnp.stack([b1, b2]).reshape(2, 1, HIDDEN)
    fcWs = jnp.stack([fc1_W, fc2_W])
    fcbs = jnp.stack([fc1_b, fc2_b]).reshape(2, 1, OUT_DIM)

    return _heads(accum, denom, batch_f, biases, fcWs, fcbs,
                  final_W, final_b.reshape(1, 1))
